# baseline (speedup 1.0000x reference)
import jax
import jax.numpy as jnp
from jax import lax
from jax.experimental import pallas as pl
from jax.experimental.pallas import tpu as pltpu
from jax.experimental.pallas import tpu_sc as plsc

N = 2048
E = 65536
H = 128
NREL = 8
NBAS = 4

NC = 2
NS = 16
NW = NC * NS
LANES = 16
EPW = E // NW
CH = 128
NCHUNK = EPW // CH


NBUF = 3


def _msgpass_body(table, gidx, didx, wnorm, zeros, out,
                  idx2, dst2, r0, r1, r2, n0, n1, n2, acc,
                  g0, g1, g2, m0, m1, m2, s0, s1, s2):
    rows = (r0, r1, r2)
    nrms = (n0, n1, n2)
    gsems = (g0, g1, g2)
    nsems = (m0, m1, m2)
    ssems = (s0, s1, s2)
    c = lax.axis_index("c")
    s = lax.axis_index("s")
    wid = s * NC + c

    pltpu.sync_copy(zeros.at[pl.ds(s * (N // NS), N // NS)],
                    acc.at[pl.ds(s * (N // NS), N // NS)])
    pltpu.sync_copy(gidx.at[wid], idx2)
    pltpu.sync_copy(didx.at[wid], dst2)
    plsc.subcore_barrier()

    def scale(b):
        @plsc.parallel_loop(0, CH, 1, unroll=4)
        def _(i):
            nv = nrms[b][i, :]
            for j in range(H // LANES):
                rows[b][i, pl.ds(j * LANES, LANES)] = (
                    rows[b][i, pl.ds(j * LANES, LANES)] * nv)

    gdesc = {}
    ndesc = {}
    sdesc = {}

    def issue(u):
        bu = u % NBUF
        gdesc[u] = pltpu.async_copy(table.at[idx2.at[u]], rows[bu],
                                    gsems[bu])
        ndesc[u] = pltpu.async_copy(wnorm.at[wid, u], nrms[bu], nsems[bu])

    for t in range(NBUF):
        issue(t)
    for t in range(NCHUNK):
        b = t % NBUF
        u = t + 1
        if NBUF <= u < NCHUNK:
            sdesc[u - NBUF].wait()
            issue(u)
        gdesc[t].wait()
        ndesc[t].wait()
        scale(b)
        sdesc[t] = pltpu.async_copy(rows[b], acc.at[dst2.at[t]], ssems[b],
                                    add=True)
    for t in range(NCHUNK - NBUF, NCHUNK):
        sdesc[t].wait()

    plsc.subcore_barrier()
    pltpu.sync_copy(acc.at[pl.ds(s * (N // NS), N // NS)],
                    out.at[pl.ds(c * N + s * (N // NS), N // NS)])


def _msgpass(table, gidx, didx, wnorm, zeros):
    mesh = plsc.VectorSubcoreMesh(core_axis_name="c", subcore_axis_name="s",
                                  num_cores=NC, num_subcores=NS)
    return pl.kernel(
        _msgpass_body,
        out_type=jax.ShapeDtypeStruct((NC * N, H), jnp.float32),
        mesh=mesh,
        scratch_types=[
            pltpu.VMEM((NCHUNK, CH), jnp.int32),
            pltpu.VMEM((NCHUNK, CH), jnp.int32),
        ] + [pltpu.VMEM((CH, H), jnp.float32) for _ in range(NBUF)]
        + [pltpu.VMEM((CH, LANES), jnp.float32) for _ in range(NBUF)]
        + [pltpu.VMEM_SHARED((N, H), jnp.float32)]
        + [pltpu.SemaphoreType.DMA for _ in range(3 * NBUF)],
    )(table, gidx, didx, wnorm, zeros)


def _table_body(x_ref, v_ref, a_ref, src_ref, rel_ref, out_ref, gidx_ref,
                zero_ref):
    x = x_ref[...]
    for r in range(NREL):
        w = a_ref[r, 0] * v_ref[0]
        for b in range(1, NBAS):
            w = w + a_ref[r, b] * v_ref[b]
        out_ref[r] = jnp.dot(x, w, preferred_element_type=jnp.float32)
    gidx_ref[...] = rel_ref[...] * N + src_ref[...]
    zero_ref[...] = jnp.zeros((N, H), jnp.float32)


def _build_table(x, v, a, src2, rel2):
    table, gidx, zeros = pl.pallas_call(
        _table_body,
        out_shape=(jax.ShapeDtypeStruct((NREL, N, H), jnp.float32),
                   jax.ShapeDtypeStruct((E // CH, CH), jnp.int32),
                   jax.ShapeDtypeStruct((N, H), jnp.float32)),
        in_specs=[
            pl.BlockSpec(memory_space=pltpu.VMEM),
            pl.BlockSpec(memory_space=pltpu.VMEM),
            pl.BlockSpec(memory_space=pltpu.SMEM),
            pl.BlockSpec(memory_space=pltpu.VMEM),
            pl.BlockSpec(memory_space=pltpu.VMEM),
        ],
        out_specs=(pl.BlockSpec(memory_space=pltpu.VMEM),
                   pl.BlockSpec(memory_space=pltpu.VMEM),
                   pl.BlockSpec(memory_space=pltpu.VMEM)),
    )(x, v, a, src2, rel2)
    return (table.reshape(NREL * N, H), gidx.reshape(NW, NCHUNK, CH), zeros)


def _act_table_body(part_ref, x_ref, wl_ref, b_ref, v_ref, a_ref,
                    h_ref, out_ref):
    p = part_ref[...]
    agg = p[:N] + p[N:]
    hnew = jax.nn.relu(agg + jnp.dot(x_ref[...], wl_ref[...],
                                     preferred_element_type=jnp.float32)
                       + b_ref[...])
    h_ref[...] = hnew
    for r in range(NREL):
        w = a_ref[r, 0] * v_ref[0]
        for b in range(1, NBAS):
            w = w + a_ref[r, b] * v_ref[b]
        out_ref[r] = jnp.dot(hnew, w, preferred_element_type=jnp.float32)


def _act_and_table(part, x, wl, bias, v, a):
    h_new, table = pl.pallas_call(
        _act_table_body,
        out_shape=(jax.ShapeDtypeStruct((N, H), jnp.float32),
                   jax.ShapeDtypeStruct((NREL, N, H), jnp.float32)),
        in_specs=[
            pl.BlockSpec(memory_space=pltpu.VMEM),
            pl.BlockSpec(memory_space=pltpu.VMEM),
            pl.BlockSpec(memory_space=pltpu.VMEM),
            pl.BlockSpec(memory_space=pltpu.VMEM),
            pl.BlockSpec(memory_space=pltpu.VMEM),
            pl.BlockSpec(memory_space=pltpu.SMEM),
        ],
        out_specs=(pl.BlockSpec(memory_space=pltpu.VMEM),
                   pl.BlockSpec(memory_space=pltpu.VMEM)),
    )(part, x, wl, bias, v, a)
    return h_new, table.reshape(NREL * N, H)


def _rownorm(u):
    nrm = jnp.sqrt(jnp.sum(u * u, axis=-1, keepdims=True))
    return u / (nrm + 1e-8)


def _head_body(part_ref, h1_ref, wl_ref, b2_ref, wm_ref, bm_ref, ws_ref,
               bs_ref, eps_ref, x0_ref, wu1_ref, bu1_ref, wu2_ref, bu2_ref,
               cw1_ref, cb1_ref, hn2_ref):
    p = part_ref[...]
    agg = p[:N] + p[N:]
    h2 = jax.nn.sigmoid(agg + jnp.dot(h1_ref[...], wl_ref[...],
                                      preferred_element_type=jnp.float32)
                        + b2_ref[...])
    z = (jnp.dot(h2, wm_ref[...], preferred_element_type=jnp.float32)
         + bm_ref[...]) + (jnp.dot(h2, ws_ref[...],
                                   preferred_element_type=jnp.float32)
                           + bs_ref[...]) * eps_ref[...]
    zn = _rownorm(z)
    g0 = jax.nn.sigmoid(lax.dot_general(zn, zn, (((1,), (1,)), ((), ())),
                                        preferred_element_type=jnp.float32))
    t0 = jnp.dot(g0, z, preferred_element_type=jnp.float32)
    hh1 = jax.nn.relu(jnp.dot(t0, wu1_ref[...],
                              preferred_element_type=jnp.float32)
                      + bu1_ref[...] + x0_ref[...])
    hn1 = _rownorm(hh1)
    g1 = jax.nn.sigmoid(lax.dot_general(hn1, hn1, (((1,), (1,)), ((), ())),
                                        preferred_element_type=jnp.float32))
    t1 = jnp.dot(g1, z, preferred_element_type=jnp.float32)
    zs = jnp.sum(z, axis=0, keepdims=True)
    u = jnp.dot(t1, wu2_ref[...], preferred_element_type=jnp.float32)
    u0 = jnp.dot(zs, wu2_ref[...], preferred_element_type=jnp.float32)
    for c in range(NBAS):
        hh2 = jax.nn.relu(cw1_ref[c, 0] * u + cb1_ref[c, 0] * u0
                          + bu2_ref[...] + h2)
        hn2_ref[c] = _rownorm(hh2)


def _decoder_head(part, h1, wl2, b2, wm, bm, ws, bs, eps, x0,
                  wu1, bu1, wu2, bu2, cw1, cb1):
    return pl.pallas_call(
        _head_body,
        out_shape=jax.ShapeDtypeStruct((NBAS, N, H), jnp.float32),
        in_specs=[pl.BlockSpec(memory_space=pltpu.VMEM)] * 14
        + [pl.BlockSpec(memory_space=pltpu.SMEM)] * 2,
        out_specs=pl.BlockSpec(memory_space=pltpu.VMEM),
    )(part, h1, wl2, b2, wm, bm, ws, bs, eps, x0, wu1, bu1, wu2, bu2,
      cw1, cb1)


TN = 128


def _final_body(hn2_ref, hn2blk_ref, cw2_ref, cb2_ref, out_ref):
    shifted = []
    for c in range(NBAS):
        g = lax.dot_general(hn2blk_ref[c], hn2_ref[c],
                            (((1,), (1,)), ((), ())),
                            preferred_element_type=jnp.float32)
        sg = jax.nn.sigmoid(g)
        zcol = jnp.zeros((TN, 1), jnp.float32)
        s_r = jnp.concatenate([zcol, sg[:, :N - 1]], axis=1)
        s_l = jnp.concatenate([sg[:, 1:], zcol], axis=1)
        shifted.append((s_r, sg, s_l))
    for r in range(NREL):
        acc = jnp.full((TN, N), cb2_ref[r, 0], jnp.float32)
        for c in range(NBAS):
            s_r, sg, s_l = shifted[c]
            acc = (acc + cw2_ref[r, 3 * c] * s_r
                   + cw2_ref[r, 3 * c + 1] * sg
                   + cw2_ref[r, 3 * c + 2] * s_l)
        out_ref[r] = acc


def _final(hn2, cw2, cb2):
    return pl.pallas_call(
        _final_body,
        grid=(N // TN,),
        out_shape=jax.ShapeDtypeStruct((NREL, N, N), jnp.float32),
        in_specs=[
            pl.BlockSpec((NBAS, N, H), lambda i: (0, 0, 0)),
            pl.BlockSpec((NBAS, TN, H), lambda i: (0, i, 0)),
            pl.BlockSpec(memory_space=pltpu.SMEM),
            pl.BlockSpec(memory_space=pltpu.SMEM),
        ],
        out_specs=pl.BlockSpec((NREL, TN, N), lambda i: (0, i, 0)),
    )(hn2, hn2, cw2, cb2)


def kernel(h, edge_index, r, norm, emb, V1, a1, Wl1, b1, V2, a2, Wl2, b2,
           Wm, bm, Ws, bs, eps, Wu1, bu1, Wu2, bu2, cw1, cb1, cw2, cb2):
    x0 = jnp.take(emb, h, axis=0)
    src2 = edge_index[0].reshape(E // CH, CH)
    rel2 = r.reshape(E // CH, CH)
    dst3 = edge_index[1].reshape(NW, NCHUNK, CH)
    normb = jnp.broadcast_to(norm[:, None],
                             (E, LANES)).reshape(NW, NCHUNK, CH, LANES)

    table1, gidx, zeros = _build_table(x0, V1, a1, src2, rel2)
    part1 = _msgpass(table1, gidx, dst3, normb, zeros)
    h1, table2 = _act_and_table(part1, x0, Wl1, b1.reshape(1, H), V2, a2)
    part2 = _msgpass(table2, gidx, dst3, normb, zeros)
    hn2 = _decoder_head(part2, h1, Wl2, b2.reshape(1, H), Wm,
                        bm.reshape(1, H), Ws, bs.reshape(1, H), eps, x0,
                        Wu1, bu1.reshape(1, H), Wu2, bu2.reshape(1, H),
                        cw1.reshape(NBAS, 1), cb1.reshape(NBAS, 1))
    recon = _final(hn2, cw2.reshape(NREL, NBAS * 3), cb2.reshape(NREL, 1))
    return recon


# speedup vs baseline: 1.0125x; 1.0125x over previous
import jax
import jax.numpy as jnp
from jax import lax
from jax.experimental import pallas as pl
from jax.experimental.pallas import tpu as pltpu
from jax.experimental.pallas import tpu_sc as plsc

N = 2048
E = 65536
H = 128
NREL = 8
NBAS = 4

NC = 2
NS = 16
NW = NC * NS
LANES = 16
EPW = E // NW
CH = 128
NCHUNK = EPW // CH


NBUF = 3


def _msgpass_body(table, gidx, didx, wnorm, zeros, out,
                  idx2, dst2, r0, r1, r2, n0, n1, n2, acc,
                  g0, g1, g2, m0, m1, m2, s0, s1, s2):
    rows = (r0, r1, r2)
    nrms = (n0, n1, n2)
    gsems = (g0, g1, g2)
    nsems = (m0, m1, m2)
    ssems = (s0, s1, s2)
    c = lax.axis_index("c")
    s = lax.axis_index("s")
    wid = s * NC + c

    pltpu.sync_copy(zeros.at[pl.ds(s * (N // NS), N // NS)],
                    acc.at[pl.ds(s * (N // NS), N // NS)])
    pltpu.sync_copy(gidx.at[wid], idx2)
    pltpu.sync_copy(didx.at[wid], dst2)
    plsc.subcore_barrier()

    def scale(b):
        def body(i, carry):
            nv = nrms[b][i, :]
            for j in range(H // LANES):
                rows[b][i, pl.ds(j * LANES, LANES)] = (
                    rows[b][i, pl.ds(j * LANES, LANES)] * nv)
            return carry
        lax.fori_loop(0, CH, body, 0)

    gdesc = {}
    ndesc = {}
    sdesc = {}

    def issue(u):
        bu = u % NBUF
        gdesc[u] = pltpu.async_copy(table.at[idx2.at[u]], rows[bu],
                                    gsems[bu])
        ndesc[u] = pltpu.async_copy(wnorm.at[wid, u], nrms[bu], nsems[bu])

    for t in range(NBUF):
        issue(t)
    for t in range(NCHUNK):
        b = t % NBUF
        u = t + 1
        if NBUF <= u < NCHUNK:
            sdesc[u - NBUF].wait()
            issue(u)
        gdesc[t].wait()
        ndesc[t].wait()
        scale(b)
        sdesc[t] = pltpu.async_copy(rows[b], acc.at[dst2.at[t]], ssems[b],
                                    add=True)
    for t in range(NCHUNK - NBUF, NCHUNK):
        sdesc[t].wait()

    plsc.subcore_barrier()
    pltpu.sync_copy(acc.at[pl.ds(s * (N // NS), N // NS)],
                    out.at[pl.ds(c * N + s * (N // NS), N // NS)])


def _msgpass(table, gidx, didx, wnorm, zeros):
    mesh = plsc.VectorSubcoreMesh(core_axis_name="c", subcore_axis_name="s",
                                  num_cores=NC, num_subcores=NS)
    return pl.kernel(
        _msgpass_body,
        out_type=jax.ShapeDtypeStruct((NC * N, H), jnp.float32),
        mesh=mesh,
        scratch_types=[
            pltpu.VMEM((NCHUNK, CH), jnp.int32),
            pltpu.VMEM((NCHUNK, CH), jnp.int32),
        ] + [pltpu.VMEM((CH, H), jnp.float32) for _ in range(NBUF)]
        + [pltpu.VMEM((CH, LANES), jnp.float32) for _ in range(NBUF)]
        + [pltpu.VMEM_SHARED((N, H), jnp.float32)]
        + [pltpu.SemaphoreType.DMA for _ in range(3 * NBUF)],
    )(table, gidx, didx, wnorm, zeros)


def _table_body(x_ref, v_ref, a_ref, src_ref, rel_ref, out_ref, gidx_ref,
                zero_ref):
    x = x_ref[...]
    for r in range(NREL):
        w = a_ref[r, 0] * v_ref[0]
        for b in range(1, NBAS):
            w = w + a_ref[r, b] * v_ref[b]
        out_ref[r] = jnp.dot(x, w, preferred_element_type=jnp.float32)
    gidx_ref[...] = rel_ref[...] * N + src_ref[...]
    zero_ref[...] = jnp.zeros((N, H), jnp.float32)


def _build_table(x, v, a, src2, rel2):
    table, gidx, zeros = pl.pallas_call(
        _table_body,
        out_shape=(jax.ShapeDtypeStruct((NREL, N, H), jnp.float32),
                   jax.ShapeDtypeStruct((E // CH, CH), jnp.int32),
                   jax.ShapeDtypeStruct((N, H), jnp.float32)),
        in_specs=[
            pl.BlockSpec(memory_space=pltpu.VMEM),
            pl.BlockSpec(memory_space=pltpu.VMEM),
            pl.BlockSpec(memory_space=pltpu.SMEM),
            pl.BlockSpec(memory_space=pltpu.VMEM),
            pl.BlockSpec(memory_space=pltpu.VMEM),
        ],
        out_specs=(pl.BlockSpec(memory_space=pltpu.VMEM),
                   pl.BlockSpec(memory_space=pltpu.VMEM),
                   pl.BlockSpec(memory_space=pltpu.VMEM)),
    )(x, v, a, src2, rel2)
    return (table.reshape(NREL * N, H), gidx.reshape(NW, NCHUNK, CH), zeros)


def _act_table_body(part_ref, x_ref, wl_ref, b_ref, v_ref, a_ref,
                    h_ref, out_ref):
    p = part_ref[...]
    agg = p[:N] + p[N:]
    hnew = jax.nn.relu(agg + jnp.dot(x_ref[...], wl_ref[...],
                                     preferred_element_type=jnp.float32)
                       + b_ref[...])
    h_ref[...] = hnew
    for r in range(NREL):
        w = a_ref[r, 0] * v_ref[0]
        for b in range(1, NBAS):
            w = w + a_ref[r, b] * v_ref[b]
        out_ref[r] = jnp.dot(hnew, w, preferred_element_type=jnp.float32)


def _act_and_table(part, x, wl, bias, v, a):
    h_new, table = pl.pallas_call(
        _act_table_body,
        out_shape=(jax.ShapeDtypeStruct((N, H), jnp.float32),
                   jax.ShapeDtypeStruct((NREL, N, H), jnp.float32)),
        in_specs=[
            pl.BlockSpec(memory_space=pltpu.VMEM),
            pl.BlockSpec(memory_space=pltpu.VMEM),
            pl.BlockSpec(memory_space=pltpu.VMEM),
            pl.BlockSpec(memory_space=pltpu.VMEM),
            pl.BlockSpec(memory_space=pltpu.VMEM),
            pl.BlockSpec(memory_space=pltpu.SMEM),
        ],
        out_specs=(pl.BlockSpec(memory_space=pltpu.VMEM),
                   pl.BlockSpec(memory_space=pltpu.VMEM)),
    )(part, x, wl, bias, v, a)
    return h_new, table.reshape(NREL * N, H)


def _rownorm(u):
    nrm = jnp.sqrt(jnp.sum(u * u, axis=-1, keepdims=True))
    return u / (nrm + 1e-8)


def _head_body(part_ref, h1_ref, wl_ref, b2_ref, wm_ref, bm_ref, ws_ref,
               bs_ref, eps_ref, x0_ref, wu1_ref, bu1_ref, wu2_ref, bu2_ref,
               cw1_ref, cb1_ref, hn2_ref):
    p = part_ref[...]
    agg = p[:N] + p[N:]
    h2 = jax.nn.sigmoid(agg + jnp.dot(h1_ref[...], wl_ref[...],
                                      preferred_element_type=jnp.float32)
                        + b2_ref[...])
    z = (jnp.dot(h2, wm_ref[...], preferred_element_type=jnp.float32)
         + bm_ref[...]) + (jnp.dot(h2, ws_ref[...],
                                   preferred_element_type=jnp.float32)
                           + bs_ref[...]) * eps_ref[...]
    zn = _rownorm(z)
    g0 = jax.nn.sigmoid(lax.dot_general(zn, zn, (((1,), (1,)), ((), ())),
                                        preferred_element_type=jnp.float32))
    t0 = jnp.dot(g0, z, preferred_element_type=jnp.float32)
    hh1 = jax.nn.relu(jnp.dot(t0, wu1_ref[...],
                              preferred_element_type=jnp.float32)
                      + bu1_ref[...] + x0_ref[...])
    hn1 = _rownorm(hh1)
    g1 = jax.nn.sigmoid(lax.dot_general(hn1, hn1, (((1,), (1,)), ((), ())),
                                        preferred_element_type=jnp.float32))
    t1 = jnp.dot(g1, z, preferred_element_type=jnp.float32)
    zs = jnp.sum(z, axis=0, keepdims=True)
    u = jnp.dot(t1, wu2_ref[...], preferred_element_type=jnp.float32)
    u0 = jnp.dot(zs, wu2_ref[...], preferred_element_type=jnp.float32)
    for c in range(NBAS):
        hh2 = jax.nn.relu(cw1_ref[c, 0] * u + cb1_ref[c, 0] * u0
                          + bu2_ref[...] + h2)
        hn2_ref[c] = _rownorm(hh2)


def _decoder_head(part, h1, wl2, b2, wm, bm, ws, bs, eps, x0,
                  wu1, bu1, wu2, bu2, cw1, cb1):
    return pl.pallas_call(
        _head_body,
        out_shape=jax.ShapeDtypeStruct((NBAS, N, H), jnp.float32),
        in_specs=[pl.BlockSpec(memory_space=pltpu.VMEM)] * 14
        + [pl.BlockSpec(memory_space=pltpu.SMEM)] * 2,
        out_specs=pl.BlockSpec(memory_space=pltpu.VMEM),
    )(part, h1, wl2, b2, wm, bm, ws, bs, eps, x0, wu1, bu1, wu2, bu2,
      cw1, cb1)


TN = 256


def _final_body(hn2_ref, hn2blk_ref, cw2_ref, cb2_ref, out_ref):
    shifted = []
    for c in range(NBAS):
        g = lax.dot_general(hn2blk_ref[c], hn2_ref[c],
                            (((1,), (1,)), ((), ())),
                            preferred_element_type=jnp.float32)
        sg = jax.nn.sigmoid(g)
        zcol = jnp.zeros((TN, 1), jnp.float32)
        s_r = jnp.concatenate([zcol, sg[:, :N - 1]], axis=1)
        s_l = jnp.concatenate([sg[:, 1:], zcol], axis=1)
        shifted.append((s_r, sg, s_l))
    for r in range(NREL):
        acc = jnp.full((TN, N), cb2_ref[r, 0], jnp.float32)
        for c in range(NBAS):
            s_r, sg, s_l = shifted[c]
            acc = (acc + cw2_ref[r, 3 * c] * s_r
                   + cw2_ref[r, 3 * c + 1] * sg
                   + cw2_ref[r, 3 * c + 2] * s_l)
        out_ref[r] = acc


def _final(hn2, cw2, cb2):
    return pl.pallas_call(
        _final_body,
        grid=(N // TN,),
        out_shape=jax.ShapeDtypeStruct((NREL, N, N), jnp.float32),
        in_specs=[
            pl.BlockSpec((NBAS, N, H), lambda i: (0, 0, 0)),
            pl.BlockSpec((NBAS, TN, H), lambda i: (0, i, 0)),
            pl.BlockSpec(memory_space=pltpu.SMEM),
            pl.BlockSpec(memory_space=pltpu.SMEM),
        ],
        out_specs=pl.BlockSpec((NREL, TN, N), lambda i: (0, i, 0)),
    )(hn2, hn2, cw2, cb2)


def kernel(h, edge_index, r, norm, emb, V1, a1, Wl1, b1, V2, a2, Wl2, b2,
           Wm, bm, Ws, bs, eps, Wu1, bu1, Wu2, bu2, cw1, cb1, cw2, cb2):
    x0 = jnp.take(emb, h, axis=0)
    src2 = edge_index[0].reshape(E // CH, CH)
    rel2 = r.reshape(E // CH, CH)
    dst3 = edge_index[1].reshape(NW, NCHUNK, CH)
    normb = jnp.broadcast_to(norm[:, None],
                             (E, LANES)).reshape(NW, NCHUNK, CH, LANES)

    table1, gidx, zeros = _build_table(x0, V1, a1, src2, rel2)
    part1 = _msgpass(table1, gidx, dst3, normb, zeros)
    h1, table2 = _act_and_table(part1, x0, Wl1, b1.reshape(1, H), V2, a2)
    part2 = _msgpass(table2, gidx, dst3, normb, zeros)
    hn2 = _decoder_head(part2, h1, Wl2, b2.reshape(1, H), Wm,
                        bm.reshape(1, H), Ws, bs.reshape(1, H), eps, x0,
                        Wu1, bu1.reshape(1, H), Wu2, bu2.reshape(1, H),
                        cw1.reshape(NBAS, 1), cb1.reshape(NBAS, 1))
    recon = _final(hn2, cw2.reshape(NREL, NBAS * 3), cb2.reshape(NREL, 1))
    return recon


# speedup vs baseline: 1.0321x; 1.0193x over previous
import jax
import jax.numpy as jnp
from jax import lax
from jax.experimental import pallas as pl
from jax.experimental.pallas import tpu as pltpu
from jax.experimental.pallas import tpu_sc as plsc

N = 2048
E = 65536
H = 128
NREL = 8
NBAS = 4

NC = 2
NS = 16
NW = NC * NS
LANES = 16
EPW = E // NW
CH = 128
NCHUNK = EPW // CH


NBUF = 3


def _msgpass_body(table, gidx, didx, wnorm, zeros, out,
                  idx2, dst2, r0, r1, r2, n0, n1, n2, acc,
                  g0, g1, g2, m0, m1, m2, s0, s1, s2):
    rows = (r0, r1, r2)
    nrms = (n0, n1, n2)
    gsems = (g0, g1, g2)
    nsems = (m0, m1, m2)
    ssems = (s0, s1, s2)
    c = lax.axis_index("c")
    s = lax.axis_index("s")
    wid = s * NC + c

    pltpu.sync_copy(gidx.at[wid], idx2)
    pltpu.sync_copy(didx.at[wid], dst2)

    def scale(b):
        def body(i, carry):
            nv = nrms[b][i, :]
            for j in range(H // LANES):
                rows[b][i, pl.ds(j * LANES, LANES)] = (
                    rows[b][i, pl.ds(j * LANES, LANES)] * nv)
            return carry
        lax.fori_loop(0, CH, body, 0)

    gdesc = {}
    ndesc = {}
    sdesc = {}

    def issue(u):
        bu = u % NBUF
        gdesc[u] = pltpu.async_copy(table.at[idx2.at[u]], rows[bu],
                                    gsems[bu])
        ndesc[u] = pltpu.async_copy(wnorm.at[wid, u], nrms[bu], nsems[bu])

    for t in range(NBUF):
        issue(t)
    pltpu.sync_copy(zeros.at[pl.ds(s * (N // NS), N // NS)],
                    acc.at[pl.ds(s * (N // NS), N // NS)])
    plsc.subcore_barrier()
    for t in range(NCHUNK):
        b = t % NBUF
        u = t + 1
        if NBUF <= u < NCHUNK:
            sdesc[u - NBUF].wait()
            issue(u)
        gdesc[t].wait()
        ndesc[t].wait()
        scale(b)
        sdesc[t] = pltpu.async_copy(rows[b], acc.at[dst2.at[t]], ssems[b],
                                    add=True)
    for t in range(NCHUNK - NBUF, NCHUNK):
        sdesc[t].wait()

    plsc.subcore_barrier()
    pltpu.sync_copy(acc.at[pl.ds(s * (N // NS), N // NS)],
                    out.at[pl.ds(c * N + s * (N // NS), N // NS)])


def _msgpass(table, gidx, didx, wnorm, zeros):
    mesh = plsc.VectorSubcoreMesh(core_axis_name="c", subcore_axis_name="s",
                                  num_cores=NC, num_subcores=NS)
    return pl.kernel(
        _msgpass_body,
        out_type=jax.ShapeDtypeStruct((NC * N, H), jnp.float32),
        mesh=mesh,
        scratch_types=[
            pltpu.VMEM((NCHUNK, CH), jnp.int32),
            pltpu.VMEM((NCHUNK, CH), jnp.int32),
        ] + [pltpu.VMEM((CH, H), jnp.float32) for _ in range(NBUF)]
        + [pltpu.VMEM((CH, LANES), jnp.float32) for _ in range(NBUF)]
        + [pltpu.VMEM_SHARED((N, H), jnp.float32)]
        + [pltpu.SemaphoreType.DMA for _ in range(3 * NBUF)],
    )(table, gidx, didx, wnorm, zeros)


def _table_body(x_ref, v_ref, a_ref, src_ref, rel_ref, out_ref, gidx_ref,
                zero_ref):
    x = x_ref[...]
    for r in range(NREL):
        w = a_ref[r, 0] * v_ref[0]
        for b in range(1, NBAS):
            w = w + a_ref[r, b] * v_ref[b]
        out_ref[r] = jnp.dot(x, w, preferred_element_type=jnp.float32)
    gidx_ref[...] = rel_ref[...] * N + src_ref[...]
    zero_ref[...] = jnp.zeros((N, H), jnp.float32)


def _build_table(x, v, a, src2, rel2):
    table, gidx, zeros = pl.pallas_call(
        _table_body,
        out_shape=(jax.ShapeDtypeStruct((NREL, N, H), jnp.float32),
                   jax.ShapeDtypeStruct((E // CH, CH), jnp.int32),
                   jax.ShapeDtypeStruct((N, H), jnp.float32)),
        in_specs=[
            pl.BlockSpec(memory_space=pltpu.VMEM),
            pl.BlockSpec(memory_space=pltpu.VMEM),
            pl.BlockSpec(memory_space=pltpu.SMEM),
            pl.BlockSpec(memory_space=pltpu.VMEM),
            pl.BlockSpec(memory_space=pltpu.VMEM),
        ],
        out_specs=(pl.BlockSpec(memory_space=pltpu.VMEM),
                   pl.BlockSpec(memory_space=pltpu.VMEM),
                   pl.BlockSpec(memory_space=pltpu.VMEM)),
    )(x, v, a, src2, rel2)
    return (table.reshape(NREL * N, H), gidx.reshape(NW, NCHUNK, CH), zeros)


def _act_table_body(part_ref, x_ref, wl_ref, b_ref, v_ref, a_ref,
                    h_ref, out_ref):
    p = part_ref[...]
    agg = p[:N] + p[N:]
    hnew = jax.nn.relu(agg + jnp.dot(x_ref[...], wl_ref[...],
                                     preferred_element_type=jnp.float32)
                       + b_ref[...])
    h_ref[...] = hnew
    for r in range(NREL):
        w = a_ref[r, 0] * v_ref[0]
        for b in range(1, NBAS):
            w = w + a_ref[r, b] * v_ref[b]
        out_ref[r] = jnp.dot(hnew, w, preferred_element_type=jnp.float32)


def _act_and_table(part, x, wl, bias, v, a):
    h_new, table = pl.pallas_call(
        _act_table_body,
        out_shape=(jax.ShapeDtypeStruct((N, H), jnp.float32),
                   jax.ShapeDtypeStruct((NREL, N, H), jnp.float32)),
        in_specs=[
            pl.BlockSpec(memory_space=pltpu.VMEM),
            pl.BlockSpec(memory_space=pltpu.VMEM),
            pl.BlockSpec(memory_space=pltpu.VMEM),
            pl.BlockSpec(memory_space=pltpu.VMEM),
            pl.BlockSpec(memory_space=pltpu.VMEM),
            pl.BlockSpec(memory_space=pltpu.SMEM),
        ],
        out_specs=(pl.BlockSpec(memory_space=pltpu.VMEM),
                   pl.BlockSpec(memory_space=pltpu.VMEM)),
    )(part, x, wl, bias, v, a)
    return h_new, table.reshape(NREL * N, H)


def _rownorm(u):
    nrm = jnp.sqrt(jnp.sum(u * u, axis=-1, keepdims=True))
    return u / (nrm + 1e-8)


TN = 128


def _dec_body(part_ref, h1_ref, wl_ref, b2_ref, wm_ref, bm_ref, ws_ref,
              bs_ref, eps_ref, x0_ref, wu1_ref, bu1_ref, wu2_ref, bu2_ref,
              cw1_ref, cb1_ref, cw2_ref, cb2_ref, out_ref, hn2_scr):
    i = pl.program_id(0)

    @pl.when(i == 0)
    def _head():
        p = part_ref[...]
        agg = p[:N] + p[N:]
        h2 = jax.nn.sigmoid(agg + jnp.dot(h1_ref[...], wl_ref[...],
                                          preferred_element_type=jnp.float32)
                            + b2_ref[...])
        z = (jnp.dot(h2, wm_ref[...], preferred_element_type=jnp.float32)
             + bm_ref[...]) + (jnp.dot(h2, ws_ref[...],
                                       preferred_element_type=jnp.float32)
                               + bs_ref[...]) * eps_ref[...]
        zn = _rownorm(z)
        g0 = jax.nn.sigmoid(
            lax.dot_general(zn, zn, (((1,), (1,)), ((), ())),
                            preferred_element_type=jnp.float32))
        t0 = jnp.dot(g0, z, preferred_element_type=jnp.float32)
        hh1 = jax.nn.relu(jnp.dot(t0, wu1_ref[...],
                                  preferred_element_type=jnp.float32)
                          + bu1_ref[...] + x0_ref[...])
        hn1 = _rownorm(hh1)
        g1 = jax.nn.sigmoid(
            lax.dot_general(hn1, hn1, (((1,), (1,)), ((), ())),
                            preferred_element_type=jnp.float32))
        t1 = jnp.dot(g1, z, preferred_element_type=jnp.float32)
        zs = jnp.sum(z, axis=0, keepdims=True)
        u = jnp.dot(t1, wu2_ref[...], preferred_element_type=jnp.float32)
        u0 = jnp.dot(zs, wu2_ref[...], preferred_element_type=jnp.float32)
        for c in range(NBAS):
            hh2 = jax.nn.relu(cw1_ref[c, 0] * u + cb1_ref[c, 0] * u0
                              + bu2_ref[...] + h2)
            hn2_scr[c] = _rownorm(hh2)

    shifted = []
    for c in range(NBAS):
        g = lax.dot_general(hn2_scr[c, pl.ds(i * TN, TN), :], hn2_scr[c],
                            (((1,), (1,)), ((), ())),
                            preferred_element_type=jnp.float32)
        sg = jax.nn.sigmoid(g)
        zcol = jnp.zeros((TN, 1), jnp.float32)
        s_r = jnp.concatenate([zcol, sg[:, :N - 1]], axis=1)
        s_l = jnp.concatenate([sg[:, 1:], zcol], axis=1)
        shifted.append((s_r, sg, s_l))
    for r in range(NREL):
        acc = jnp.full((TN, N), cb2_ref[r, 0], jnp.float32)
        for c in range(NBAS):
            s_r, sg, s_l = shifted[c]
            acc = (acc + cw2_ref[r, 3 * c] * s_r
                   + cw2_ref[r, 3 * c + 1] * sg
                   + cw2_ref[r, 3 * c + 2] * s_l)
        out_ref[r] = acc


def _decoder(part, h1, wl2, b2, wm, bm, ws, bs, eps, x0,
             wu1, bu1, wu2, bu2, cw1, cb1, cw2, cb2):
    return pl.pallas_call(
        _dec_body,
        grid=(N // TN,),
        out_shape=jax.ShapeDtypeStruct((NREL, N, N), jnp.float32),
        in_specs=[pl.BlockSpec(memory_space=pltpu.VMEM)] * 14
        + [pl.BlockSpec(memory_space=pltpu.SMEM)] * 4,
        out_specs=pl.BlockSpec((NREL, TN, N), lambda i: (0, i, 0)),
        scratch_shapes=[pltpu.VMEM((NBAS, N, H), jnp.float32)],
    )(part, h1, wl2, b2, wm, bm, ws, bs, eps, x0, wu1, bu1, wu2, bu2,
      cw1, cb1, cw2, cb2)


def kernel(h, edge_index, r, norm, emb, V1, a1, Wl1, b1, V2, a2, Wl2, b2,
           Wm, bm, Ws, bs, eps, Wu1, bu1, Wu2, bu2, cw1, cb1, cw2, cb2):
    x0 = jnp.take(emb, h, axis=0)
    src2 = edge_index[0].reshape(E // CH, CH)
    rel2 = r.reshape(E // CH, CH)
    dst3 = edge_index[1].reshape(NW, NCHUNK, CH)
    normb = jnp.broadcast_to(norm[:, None],
                             (E, LANES)).reshape(NW, NCHUNK, CH, LANES)

    table1, gidx, zeros = _build_table(x0, V1, a1, src2, rel2)
    part1 = _msgpass(table1, gidx, dst3, normb, zeros)
    h1, table2 = _act_and_table(part1, x0, Wl1, b1.reshape(1, H), V2, a2)
    part2 = _msgpass(table2, gidx, dst3, normb, zeros)
    recon = _decoder(part2, h1, Wl2, b2.reshape(1, H), Wm,
                     bm.reshape(1, H), Ws, bs.reshape(1, H), eps, x0,
                     Wu1, bu1.reshape(1, H), Wu2, bu2.reshape(1, H),
                     cw1.reshape(NBAS, 1), cb1.reshape(NBAS, 1),
                     cw2.reshape(NREL, NBAS * 3), cb2.reshape(NREL, 1))
    return recon
